# baseline (speedup 1.0000x reference)
import jax
import jax.numpy as jnp
from jax.experimental import pallas as pl
from jax.experimental.pallas import tpu as pltpu

BN_EPS = 1e-5
NORM_EPS = 1e-12
CPAD = 128


def _half(x_ref, cw_ref, bias_ref, K):
    xb = x_ref[0].astype(jnp.bfloat16)
    logits = jnp.dot(xb, cw_ref[...],
                     preferred_element_type=jnp.float32) + bias_ref[...]
    e = jnp.exp(logits)
    s = jnp.sum(e, axis=-1, keepdims=True)
    a = e[:, :K] / s
    acc = jax.lax.dot_general(
        a.astype(jnp.bfloat16), xb, (((0,), (0,)), ((), ())),
        preferred_element_type=jnp.float32)
    return acc, jnp.sum(a, axis=0, keepdims=True)


def _netvlad_kernel(x1_ref, x2_ref, x3_ref, x4_ref, cl_ref, cl2_ref,
                    g_ref, b_ref, m_ref, v_ref, out_ref, cw_ref, bias_ref):
    b = pl.program_id(0)
    K = cl2_ref.shape[2]
    C = cl_ref.shape[1]

    @pl.when(b == 0)
    def _():
        scale = g_ref[...] * jax.lax.rsqrt(v_ref[...] + BN_EPS)
        bias_ref[...] = jnp.full(bias_ref.shape, -1e30, jnp.float32)
        bias_ref[:, :C] = b_ref[...] - m_ref[...] * scale
        cw_ref[...] = jnp.zeros(cw_ref.shape, jnp.bfloat16)
        cw_ref[:, :C] = (cl_ref[...] * scale).astype(jnp.bfloat16)

    acc1, asum1 = _half(x1_ref, cw_ref, bias_ref, K)
    acc2, asum2 = _half(x2_ref, cw_ref, bias_ref, K)
    acc3, asum3 = _half(x3_ref, cw_ref, bias_ref, K)
    acc4, asum4 = _half(x4_ref, cw_ref, bias_ref, K)
    acc = (acc1 + acc2) + (acc3 + acc4)
    asum = (asum1 + asum2) + (asum3 + asum4)

    vlad = acc.T - asum * cl2_ref[0]
    n1 = jnp.sqrt(jnp.sum(vlad * vlad, axis=0, keepdims=True))
    vlad = vlad / jnp.maximum(n1, NORM_EPS)
    n2 = jnp.sqrt(jnp.sum(vlad * vlad))
    vlad = vlad / jnp.maximum(n2, NORM_EPS)
    out_ref[0] = vlad


def kernel(x, clusters, clusters2, bn_gamma, bn_beta, bn_mean, bn_var):
    B, N, D = x.shape
    C = clusters.shape[1]
    K = clusters2.shape[2]

    out = pl.pallas_call(
        _netvlad_kernel,
        out_shape=jax.ShapeDtypeStruct((B, D, K), jnp.float32),
        grid=(B,),
        in_specs=[
            pl.BlockSpec((1, N // 4, D), lambda b: (b, 0, 0)),
            pl.BlockSpec((1, N // 4, D), lambda b: (b, 1, 0)),
            pl.BlockSpec((1, N // 4, D), lambda b: (b, 2, 0)),
            pl.BlockSpec((1, N // 4, D), lambda b: (b, 3, 0)),
            pl.BlockSpec((D, C), lambda b: (0, 0)),
            pl.BlockSpec((1, D, K), lambda b: (0, 0, 0)),
            pl.BlockSpec((1, C), lambda b: (0, 0)),
            pl.BlockSpec((1, C), lambda b: (0, 0)),
            pl.BlockSpec((1, C), lambda b: (0, 0)),
            pl.BlockSpec((1, C), lambda b: (0, 0)),
        ],
        out_specs=pl.BlockSpec((1, D, K), lambda b: (b, 0, 0)),
        scratch_shapes=[
            pltpu.VMEM((D, CPAD), jnp.bfloat16),
            pltpu.VMEM((1, CPAD), jnp.float32),
        ],
        compiler_params=pltpu.CompilerParams(
            dimension_semantics=("arbitrary",),
        ),
        name="netvlad_fused",
    )(x, x, x, x, clusters, clusters2,
      bn_gamma.reshape(1, C), bn_beta.reshape(1, C),
      bn_mean.reshape(1, C), bn_var.reshape(1, C))
    return out.reshape(B, D * K)


# speedup vs baseline: 1.0349x; 1.0349x over previous
import jax
import jax.numpy as jnp
from jax.experimental import pallas as pl
from jax.experimental.pallas import tpu as pltpu

BN_EPS = 1e-5
NORM_EPS = 1e-12
CPAD = 128


def _half(x_ref, cw_ref, bias_ref, K):
    xb = x_ref[0].astype(jnp.bfloat16)
    logits = jnp.dot(xb, cw_ref[...],
                     preferred_element_type=jnp.float32) + bias_ref[...]
    e = jnp.exp(logits)
    s = jnp.sum(e, axis=-1, keepdims=True)
    a = e[:, :K] / s
    acc = jax.lax.dot_general(
        a.astype(jnp.bfloat16), xb, (((0,), (0,)), ((), ())),
        preferred_element_type=jnp.float32)
    return acc, jnp.sum(a, axis=0, keepdims=True)


def _netvlad_kernel(x1_ref, x2_ref, cl_ref, cl2_ref, g_ref, b_ref, m_ref,
                    v_ref, out_ref, cw_ref, bias_ref):
    b = pl.program_id(0)
    K = cl2_ref.shape[2]
    C = cl_ref.shape[1]

    @pl.when(b == 0)
    def _():
        scale = g_ref[...] * jax.lax.rsqrt(v_ref[...] + BN_EPS)
        bias_ref[...] = jnp.full(bias_ref.shape, -1e30, jnp.float32)
        bias_ref[:, :C] = b_ref[...] - m_ref[...] * scale
        cw_ref[...] = jnp.zeros(cw_ref.shape, jnp.bfloat16)
        cw_ref[:, :C] = (cl_ref[...] * scale).astype(jnp.bfloat16)

    acc1, asum1 = _half(x1_ref, cw_ref, bias_ref, K)
    acc2, asum2 = _half(x2_ref, cw_ref, bias_ref, K)
    acc = acc1 + acc2
    asum = asum1 + asum2

    vlad = acc.T - asum * cl2_ref[0]
    n1 = jnp.sqrt(jnp.sum(vlad * vlad, axis=0, keepdims=True))
    vlad = vlad / jnp.maximum(n1, NORM_EPS)
    n2 = jnp.sqrt(jnp.sum(vlad * vlad))
    vlad = vlad / jnp.maximum(n2, NORM_EPS)
    out_ref[0] = vlad


def kernel(x, clusters, clusters2, bn_gamma, bn_beta, bn_mean, bn_var):
    B, N, D = x.shape
    C = clusters.shape[1]
    K = clusters2.shape[2]

    out = pl.pallas_call(
        _netvlad_kernel,
        out_shape=jax.ShapeDtypeStruct((B, D, K), jnp.float32),
        grid=(B,),
        in_specs=[
            pl.BlockSpec((1, N // 2, D), lambda b: (b, 0, 0)),
            pl.BlockSpec((1, N // 2, D), lambda b: (b, 1, 0)),
            pl.BlockSpec((D, C), lambda b: (0, 0)),
            pl.BlockSpec((1, D, K), lambda b: (0, 0, 0)),
            pl.BlockSpec((1, C), lambda b: (0, 0)),
            pl.BlockSpec((1, C), lambda b: (0, 0)),
            pl.BlockSpec((1, C), lambda b: (0, 0)),
            pl.BlockSpec((1, C), lambda b: (0, 0)),
        ],
        out_specs=pl.BlockSpec((1, D, K), lambda b: (b, 0, 0)),
        scratch_shapes=[
            pltpu.VMEM((D, CPAD), jnp.bfloat16),
            pltpu.VMEM((1, CPAD), jnp.float32),
        ],
        compiler_params=pltpu.CompilerParams(
            dimension_semantics=("arbitrary",),
        ),
        name="netvlad_fused",
    )(x, x, clusters, clusters2,
      bn_gamma.reshape(1, C), bn_beta.reshape(1, C),
      bn_mean.reshape(1, C), bn_var.reshape(1, C))
    return out.reshape(B, D * K)
